# Initial kernel scaffold; baseline (speedup 1.0000x reference)
import jax
import jax.numpy as jnp
from jax.experimental import pallas as pl


def kernel(coords, obses, coord_table, field_table):
    raise NotImplementedError("write your pallas kernel here")


# speedup vs baseline: 8.9377x; 8.9377x over previous
import functools

import jax
import jax.numpy as jnp
from jax import lax
from jax.experimental import pallas as pl
from jax.experimental.pallas import tpu as pltpu
from jax.experimental.pallas import tpu_sc as plsc

_BATCH = 4096
_FCHUNK = 100
_NCHUNK = 4
_ROW_W = 12928


def _sc_embed(cidx8, obsesT, ctab2, ftab):
    info = plsc.get_sparse_core_info()
    nw = info.num_cores * info.num_subcores
    rows_per_w = _BATCH // nw

    mesh = plsc.VectorSubcoreMesh(core_axis_name="c", subcore_axis_name="s")

    @functools.partial(
        pl.kernel,
        mesh=mesh,
        out_type=jax.ShapeDtypeStruct((_BATCH * 404, 32), jnp.float32),
        compiler_params=pltpu.CompilerParams(use_tc_tiling_on_sc=False),
        scratch_types=[
            pltpu.VMEM((8 * rows_per_w,), jnp.int32),
            pltpu.VMEM((_NCHUNK, _FCHUNK), jnp.int32),
            pltpu.VMEM((404, 32), jnp.float32),
            pltpu.SemaphoreType.DMA,
        ],
    )
    def k(cidx_hbm, oidx_hbm, ctab_hbm, ftab_hbm, out_hbm,
          cidx_v, oidx_v, fbuf, sem):
        cid = lax.axis_index("c")
        sid = lax.axis_index("s")
        wid = sid * info.num_cores + cid
        base = wid * rows_per_w

        pltpu.sync_copy(cidx_hbm.at[pl.ds(8 * base, 8 * rows_per_w)], cidx_v)

        def row_body(i, carry):
            row = base + i
            pltpu.sync_copy(oidx_hbm.at[row], oidx_v)
            pltpu.async_copy(
                ctab_hbm.at[cidx_v.at[pl.ds(8 * i, 8)]],
                fbuf.at[pl.ds(0, 8)], sem).wait()
            fps = [
                pltpu.async_copy(
                    ftab_hbm.at[oidx_v.at[j]],
                    fbuf.at[pl.ds(4 + _FCHUNK * j, _FCHUNK)], sem)
                for j in range(_NCHUNK)
            ]
            for p in fps:
                p.wait()
            pltpu.sync_copy(fbuf, out_hbm.at[pl.ds(row * 404, 404), :])
            return carry

        lax.fori_loop(0, rows_per_w, row_body, 0)

    return k(cidx8, obsesT, ctab2, ftab)


def kernel(coords, obses, coord_table, field_table):
    batch = coords.shape[0]
    obsesT = obses.reshape(batch, _NCHUNK, _FCHUNK)
    ctab2 = coord_table.reshape(-1, 32)
    cidx2 = (2 * coords[:, :, None]
             + jnp.arange(2, dtype=jnp.int32)[None, None, :]).reshape(batch, 4)
    cidx8 = jnp.concatenate([cidx2, cidx2], axis=1).reshape(-1)
    out = _sc_embed(cidx8, obsesT, ctab2, field_table)
    return out.reshape(batch, _ROW_W)


# speedup vs baseline: 9.3889x; 1.0505x over previous
import functools

import jax
import jax.numpy as jnp
from jax import lax
from jax.experimental import pallas as pl
from jax.experimental.pallas import tpu as pltpu
from jax.experimental.pallas import tpu_sc as plsc

_BATCH = 4096
_FCHUNK = 100
_NCHUNK = 4
_ROW_W = 12928
_NSLOT = 4
_ROW_EMB = 404


def _sc_embed(cidx2, obsesF, ctab2, ftab):
    info = plsc.get_sparse_core_info()
    nw = info.num_cores * info.num_subcores
    rows_per_w = _BATCH // nw
    pairs_per_w = rows_per_w // 2
    iters = pairs_per_w // _NSLOT

    mesh = plsc.VectorSubcoreMesh(core_axis_name="c", subcore_axis_name="s")

    @functools.partial(
        pl.kernel,
        mesh=mesh,
        out_type=jax.ShapeDtypeStruct((_BATCH * _ROW_EMB, 32), jnp.float32),
        compiler_params=pltpu.CompilerParams(use_tc_tiling_on_sc=False),
        scratch_types=[
            pltpu.VMEM((rows_per_w, 4), jnp.int32),
            [pltpu.VMEM((2 * _NCHUNK, _FCHUNK), jnp.int32)] * _NSLOT,
            [pltpu.VMEM((2 * _ROW_EMB, 32), jnp.float32)] * _NSLOT,
            [pltpu.SemaphoreType.DMA] * _NSLOT,
            [pltpu.SemaphoreType.DMA] * _NSLOT,
            [pltpu.SemaphoreType.DMA] * _NSLOT,
        ],
    )
    def k(cidx_hbm, oidx_hbm, ctab_hbm, ftab_hbm, out_hbm,
          cidx_v, oidx_s, fbuf_s, gsem_s, isem_s, wsem_s):
        cid = lax.axis_index("c")
        sid = lax.axis_index("s")
        wid = sid * info.num_cores + cid
        base = wid * rows_per_w
        pbase = wid * pairs_per_w

        pltpu.sync_copy(cidx_hbm.at[pl.ds(base, rows_per_w), :], cidx_v)
        for s in range(_NSLOT):
            pltpu.async_copy(oidx_hbm.at[pl.ds(8 * (pbase + s), 8), :],
                             oidx_s[s], isem_s[s])

        def issue_gathers(s, p):
            descs = [
                pltpu.async_copy(
                    ctab_hbm.at[cidx_v.at[2 * p + r]],
                    fbuf_s[s].at[pl.ds(r * _ROW_EMB, 4)], gsem_s[s])
                for r in range(2)
            ]
            descs += [
                pltpu.async_copy(
                    ftab_hbm.at[oidx_s[s].at[r * _NCHUNK + j]],
                    fbuf_s[s].at[pl.ds(r * _ROW_EMB + 4 + _FCHUNK * j,
                                       _FCHUNK)], gsem_s[s])
                for r in range(2) for j in range(_NCHUNK)
            ]
            return descs

        def body(t, carry):
            descs = []
            for s in range(_NSLOT):
                p = _NSLOT * t + s
                row0 = (base + 2 * p) * _ROW_EMB

                @pl.when(t > 0)
                def _():
                    pltpu.make_async_copy(
                        fbuf_s[s], out_hbm.at[pl.ds(row0, 2 * _ROW_EMB), :],
                        wsem_s[s]).wait()

                pltpu.make_async_copy(
                    oidx_hbm.at[pl.ds(8 * (pbase + p), 8), :],
                    oidx_s[s], isem_s[s]).wait()
                descs.append(issue_gathers(s, p))
            for s in range(_NSLOT):
                p = _NSLOT * t + s
                row0 = (base + 2 * p) * _ROW_EMB
                for d in descs[s]:
                    d.wait()
                pltpu.async_copy(
                    fbuf_s[s], out_hbm.at[pl.ds(row0, 2 * _ROW_EMB), :],
                    wsem_s[s])
                pn = jnp.minimum(p + _NSLOT, pairs_per_w - 1)
                pltpu.async_copy(oidx_hbm.at[pl.ds(8 * (pbase + pn), 8), :],
                                 oidx_s[s], isem_s[s])
            return carry

        lax.fori_loop(0, iters, body, 0)

        for s in range(_NSLOT):
            p = _NSLOT * (iters - 1) + s
            row0 = (base + 2 * p) * _ROW_EMB
            pltpu.make_async_copy(
                oidx_hbm.at[pl.ds(8 * (pbase + p), 8), :],
                oidx_s[s], isem_s[s]).wait()
            pltpu.make_async_copy(
                fbuf_s[s], out_hbm.at[pl.ds(row0, 2 * _ROW_EMB), :],
                wsem_s[s]).wait()

    return k(cidx2, obsesF, ctab2, ftab)


def kernel(coords, obses, coord_table, field_table):
    batch = coords.shape[0]
    obsesF = obses.reshape(batch * _NCHUNK, _FCHUNK)
    ctab2 = coord_table.reshape(-1, 32)
    cidx2 = (2 * coords[:, :, None]
             + jnp.arange(2, dtype=jnp.int32)[None, None, :]).reshape(batch, 4)
    out = _sc_embed(cidx2, obsesF, ctab2, field_table)
    return out.reshape(batch, _ROW_W)


# speedup vs baseline: 13.2459x; 1.4108x over previous
import functools

import jax
import jax.numpy as jnp
from jax import lax
from jax.experimental import pallas as pl
from jax.experimental.pallas import tpu as pltpu
from jax.experimental.pallas import tpu_sc as plsc

_BATCH = 4096
_FCHUNK = 100
_NCHUNK = 4
_ROW_W = 12928
_NSLOT = 4
_ROW_EMB = 404


def _sc_embed(cidx2, obsesF, ctab2, ftab):
    info = plsc.get_sparse_core_info()
    nw = info.num_cores * info.num_subcores
    rows_per_w = _BATCH // nw
    pairs_per_w = rows_per_w // 2
    iters = pairs_per_w // _NSLOT

    mesh = plsc.VectorSubcoreMesh(core_axis_name="c", subcore_axis_name="s")

    @functools.partial(
        pl.kernel,
        mesh=mesh,
        out_type=jax.ShapeDtypeStruct((_BATCH * _ROW_EMB, 32), jnp.float32),
        compiler_params=pltpu.CompilerParams(use_tc_tiling_on_sc=False),
        scratch_types=[
            pltpu.VMEM((rows_per_w, 4), jnp.int32),
            [pltpu.VMEM((2 * _NCHUNK, _FCHUNK), jnp.int32)] * _NSLOT,
            [pltpu.VMEM((2 * _ROW_EMB, 32), jnp.float32)] * _NSLOT,
            [pltpu.SemaphoreType.DMA] * _NSLOT,
            [pltpu.SemaphoreType.DMA] * _NSLOT,
            [pltpu.SemaphoreType.DMA] * _NSLOT,
        ],
    )
    def k(cidx_hbm, oidx_hbm, ctab_hbm, ftab_hbm, out_hbm,
          cidx_v, oidx_s, fbuf_s, gsem_s, isem_s, wsem_s):
        cid = lax.axis_index("c")
        sid = lax.axis_index("s")
        wid = sid * info.num_cores + cid
        base = wid * rows_per_w
        pbase = wid * pairs_per_w

        pltpu.sync_copy(cidx_hbm.at[pl.ds(base, rows_per_w), :], cidx_v)
        for s in range(_NSLOT):
            pltpu.async_copy(oidx_hbm.at[pl.ds(8 * (pbase + s), 8), :],
                             oidx_s[s], isem_s[s])

        def issue_gathers(s, p):
            descs = [
                pltpu.async_copy(
                    ctab_hbm.at[cidx_v.at[2 * p + r]],
                    fbuf_s[s].at[pl.ds(r * _ROW_EMB, 4)], gsem_s[s])
                for r in range(2)
            ]
            descs += [
                pltpu.async_copy(
                    ftab_hbm.at[oidx_s[s].at[r * _NCHUNK + j]],
                    fbuf_s[s].at[pl.ds(r * _ROW_EMB + 4 + _FCHUNK * j,
                                       _FCHUNK)], gsem_s[s])
                for r in range(2) for j in range(_NCHUNK)
            ]
            return descs

        def body(t, carry):
            descs = []
            for s in range(_NSLOT):
                p = _NSLOT * t + s
                row0 = (base + 2 * p) * _ROW_EMB

                @pl.when(t > 0)
                def _():
                    pltpu.make_async_copy(
                        fbuf_s[s], out_hbm.at[pl.ds(row0, 2 * _ROW_EMB), :],
                        wsem_s[s]).wait()

                pltpu.make_async_copy(
                    oidx_hbm.at[pl.ds(8 * (pbase + p), 8), :],
                    oidx_s[s], isem_s[s]).wait()
                descs.append(issue_gathers(s, p))
            for s in range(_NSLOT):
                p = _NSLOT * t + s
                row0 = (base + 2 * p) * _ROW_EMB
                for d in descs[s]:
                    d.wait()
                pltpu.async_copy(
                    fbuf_s[s], out_hbm.at[pl.ds(row0, 2 * _ROW_EMB), :],
                    wsem_s[s])
                pn = jnp.minimum(p + _NSLOT, pairs_per_w - 1)
                pltpu.async_copy(oidx_hbm.at[pl.ds(8 * (pbase + pn), 8), :],
                                 oidx_s[s], isem_s[s])
            return carry

        lax.fori_loop(0, iters, body, 0)

        for s in range(_NSLOT):
            p = _NSLOT * (iters - 1) + s
            row0 = (base + 2 * p) * _ROW_EMB
            pltpu.make_async_copy(
                oidx_hbm.at[pl.ds(8 * (pbase + p), 8), :],
                oidx_s[s], isem_s[s]).wait()
            pltpu.make_async_copy(
                fbuf_s[s], out_hbm.at[pl.ds(row0, 2 * _ROW_EMB), :],
                wsem_s[s]).wait()

    return k(cidx2, obsesF, ctab2, ftab)


def kernel(coords, obses, coord_table, field_table):
    batch = coords.shape[0]
    info = plsc.get_sparse_core_info()
    nw = info.num_cores * info.num_subcores
    rows_per_w = batch // nw
    ftabR = jnp.tile(field_table, (nw, 1))
    biasF = (jnp.arange(batch * _NCHUNK, dtype=jnp.int32)
             // (rows_per_w * _NCHUNK)) * field_table.shape[0]
    obsesF = obses.reshape(batch * _NCHUNK, _FCHUNK) + biasF[:, None]
    ctab2 = coord_table.reshape(-1, 32)
    cidx2 = (2 * coords[:, :, None]
             + jnp.arange(2, dtype=jnp.int32)[None, None, :]).reshape(batch, 4)
    out = _sc_embed(cidx2, obsesF, ctab2, ftabR)
    return out.reshape(batch, _ROW_W)


# speedup vs baseline: 13.2586x; 1.0010x over previous
import functools

import jax
import jax.numpy as jnp
from jax import lax
from jax.experimental import pallas as pl
from jax.experimental.pallas import tpu as pltpu
from jax.experimental.pallas import tpu_sc as plsc

_BATCH = 4096
_FCHUNK = 100
_NCHUNK = 4
_ROW_W = 12928
_NSLOT = 2
_SROWS = 4
_ROW_EMB = 404


def _sc_embed(cidx2, obsesF, ctab2, ftab):
    info = plsc.get_sparse_core_info()
    nw = info.num_cores * info.num_subcores
    rows_per_w = _BATCH // nw
    slots_per_w = rows_per_w // _SROWS
    iters = slots_per_w // _NSLOT

    mesh = plsc.VectorSubcoreMesh(core_axis_name="c", subcore_axis_name="s")

    @functools.partial(
        pl.kernel,
        mesh=mesh,
        out_type=jax.ShapeDtypeStruct((_BATCH * _ROW_EMB, 32), jnp.float32),
        compiler_params=pltpu.CompilerParams(use_tc_tiling_on_sc=False),
        scratch_types=[
            pltpu.VMEM((rows_per_w, 4), jnp.int32),
            [pltpu.VMEM((_SROWS * _NCHUNK, _FCHUNK), jnp.int32)] * _NSLOT,
            [pltpu.VMEM((_SROWS * _ROW_EMB, 32), jnp.float32)] * _NSLOT,
            [pltpu.SemaphoreType.DMA] * _NSLOT,
            [pltpu.SemaphoreType.DMA] * _NSLOT,
            [pltpu.SemaphoreType.DMA] * _NSLOT,
        ],
    )
    def k(cidx_hbm, oidx_hbm, ctab_hbm, ftab_hbm, out_hbm,
          cidx_v, oidx_s, fbuf_s, gsem_s, isem_s, wsem_s):
        cid = lax.axis_index("c")
        sid = lax.axis_index("s")
        wid = sid * info.num_cores + cid
        base = wid * rows_per_w
        qbase = wid * slots_per_w

        pltpu.sync_copy(cidx_hbm.at[pl.ds(base, rows_per_w), :], cidx_v)
        for s in range(_NSLOT):
            pltpu.async_copy(
                oidx_hbm.at[pl.ds(_SROWS * _NCHUNK * (qbase + s),
                                  _SROWS * _NCHUNK), :],
                oidx_s[s], isem_s[s])

        def issue_gathers(s, q):
            for r in range(_SROWS):
                pltpu.async_copy(
                    ctab_hbm.at[cidx_v.at[_SROWS * q + r]],
                    fbuf_s[s].at[pl.ds(r * _ROW_EMB, 4)], gsem_s[s])
                for j in range(_NCHUNK):
                    pltpu.async_copy(
                        ftab_hbm.at[oidx_s[s].at[r * _NCHUNK + j]],
                        fbuf_s[s].at[pl.ds(r * _ROW_EMB + 4 + _FCHUNK * j,
                                           _FCHUNK)], gsem_s[s])

        def body(t, carry):
            for s in range(_NSLOT):
                q = _NSLOT * t + s
                row0 = (base + _SROWS * q) * _ROW_EMB

                @pl.when(t > 0)
                def _():
                    pltpu.make_async_copy(
                        fbuf_s[s],
                        out_hbm.at[pl.ds(row0, _SROWS * _ROW_EMB), :],
                        wsem_s[s]).wait()

                pltpu.make_async_copy(
                    oidx_hbm.at[pl.ds(_SROWS * _NCHUNK * (qbase + q),
                                      _SROWS * _NCHUNK), :],
                    oidx_s[s], isem_s[s]).wait()
                issue_gathers(s, q)
            for s in range(_NSLOT):
                q = _NSLOT * t + s
                row0 = (base + _SROWS * q) * _ROW_EMB
                pltpu.make_async_copy(
                    out_hbm.at[pl.ds(row0, _SROWS * _ROW_EMB), :],
                    fbuf_s[s], gsem_s[s]).wait()
                pltpu.async_copy(
                    fbuf_s[s], out_hbm.at[pl.ds(row0, _SROWS * _ROW_EMB), :],
                    wsem_s[s])
                qn = jnp.minimum(q + _NSLOT, slots_per_w - 1)
                pltpu.async_copy(
                    oidx_hbm.at[pl.ds(_SROWS * _NCHUNK * (qbase + qn),
                                      _SROWS * _NCHUNK), :],
                    oidx_s[s], isem_s[s])
            return carry

        lax.fori_loop(0, iters, body, 0)

        for s in range(_NSLOT):
            q = _NSLOT * (iters - 1) + s
            row0 = (base + _SROWS * q) * _ROW_EMB
            pltpu.make_async_copy(
                oidx_hbm.at[pl.ds(_SROWS * _NCHUNK * (qbase + q),
                                  _SROWS * _NCHUNK), :],
                oidx_s[s], isem_s[s]).wait()
            pltpu.make_async_copy(
                fbuf_s[s], out_hbm.at[pl.ds(row0, _SROWS * _ROW_EMB), :],
                wsem_s[s]).wait()

    return k(cidx2, obsesF, ctab2, ftab)


def kernel(coords, obses, coord_table, field_table):
    batch = coords.shape[0]
    info = plsc.get_sparse_core_info()
    nw = info.num_cores * info.num_subcores
    rows_per_w = batch // nw
    ftabR = jnp.tile(field_table, (nw, 1))
    biasF = (jnp.arange(batch * _NCHUNK, dtype=jnp.int32)
             // (rows_per_w * _NCHUNK)) * field_table.shape[0]
    obsesF = obses.reshape(batch * _NCHUNK, _FCHUNK) + biasF[:, None]
    ctab2 = coord_table.reshape(-1, 32)
    cidx2 = (2 * coords[:, :, None]
             + jnp.arange(2, dtype=jnp.int32)[None, None, :]).reshape(batch, 4)
    out = _sc_embed(cidx2, obsesF, ctab2, ftabR)
    return out.reshape(batch, _ROW_W)
